# baseline (speedup 1.0000x reference)
import jax
import jax.numpy as jnp
from jax import lax
from jax.experimental import pallas as pl
from jax.experimental.pallas import tpu as pltpu
from jax.experimental.pallas import tpu_sc as plsc

NC = 2
NS = 16


def _score_body(x_ref, p_ref, o_ref):
    p = p_ref[0, :]
    xb = x_ref[...].astype(jnp.bfloat16)
    pb = p.astype(jnp.bfloat16)[:, None]
    s = lax.dot_general(xb, pb, (((1,), (0,)), ((), ())),
                        preferred_element_type=jnp.float32)
    o_ref[...] = s * lax.rsqrt(jnp.sum(p * p))


def _gru_w(xg, vals, w0, wih, whh, bih, bhh):
    d = w0.shape[1]
    xt = xg * jnp.tanh(vals)
    gi = lax.dot_general(xt, wih, (((1,), (1,)), ((), ())),
                         preferred_element_type=jnp.float32, precision=lax.Precision.HIGHEST) + bih
    gh = lax.dot_general(w0, whh, (((1,), (1,)), ((), ())),
                         preferred_element_type=jnp.float32, precision=lax.Precision.HIGHEST) + bhh
    r = jax.nn.sigmoid(gi[:, :d] + gh[:, :d])
    z = jax.nn.sigmoid(gi[:, d:2 * d] + gh[:, d:2 * d])
    n = jnp.tanh(gi[:, 2 * d:] + r * gh[:, 2 * d:])
    return (1.0 - z) * n + z * w0


def _xw_body(x_ref, xg_ref, vals_ref, w0_ref, wih_ref, whh_ref, bih_ref,
             bhh_ref, d0_ref, d1_ref, y0_ref, y1_ref, dinv_ref, w_scr):
    h = y0_ref.shape[1]

    @pl.when(pl.program_id(0) == 0)
    def _():
        w_scr[...] = _gru_w(xg_ref[...], vals_ref[...], w0_ref[...],
                            wih_ref[...], whh_ref[...], bih_ref[...],
                            bhh_ref[...])

    deg = d0_ref[...] + d1_ref[...] + 1.0
    dinv = lax.rsqrt(deg)
    xw = jnp.dot(x_ref[...], w_scr[...], preferred_element_type=jnp.float32, precision=lax.Precision.HIGHEST)
    y = xw * dinv
    y0_ref[...] = y[:, :h]
    y1_ref[...] = y[:, h:]
    dinv_ref[...] = dinv


def _out_body(h0_ref, h1_ref, y0_ref, y1_ref, dinv_ref, lw_ref, lb_ref,
              o_ref):
    dinv = dinv_ref[...]
    pre = jnp.concatenate(
        [h0_ref[...] + y0_ref[...], h1_ref[...] + y1_ref[...]], axis=1)
    r = jnp.maximum(pre * dinv, 0.0)
    o_ref[...] = lax.dot_general(r, lw_ref[...], (((1,), (1,)), ((), ())),
                                 preferred_element_type=jnp.float32, precision=lax.Precision.HIGHEST) + lb_ref[...]


def _sc_gather_rows_body(x_hbm, perm_hbm, xg_out, permbuf, rowsbuf, gsem):
    c = lax.axis_index("c")
    s = lax.axis_index("s")
    wid = c * NS + s
    k_per = perm_hbm.shape[0] // (NC * NS)
    pltpu.sync_copy(perm_hbm.at[pl.ds(wid * k_per, k_per)], permbuf)
    pltpu.async_copy(x_hbm.at[permbuf], rowsbuf, gsem).wait()
    pltpu.sync_copy(rowsbuf, xg_out.at[pl.ds(wid * k_per, k_per), :])


def _sc_deg_body(dst2_hbm, deg0_out, deg1_out,
                 idxbuf, idxbuf1, onesbuf, zbuf, deg_sh, dsem0, dsem1):
    c = lax.axis_index("c")
    s = lax.axis_index("s")

    def fill_ones(i, carry):
        onesbuf[pl.ds(i * 16, 16)] = jnp.full((16,), 1.0, jnp.float32)
        return carry
    lax.fori_loop(0, onesbuf.shape[0] // 16, fill_ones, 0)

    def fill_z(i, carry):
        zbuf[pl.ds(i * 16, 16)] = jnp.zeros((16,), jnp.float32)
        return carry
    lax.fori_loop(0, zbuf.shape[0] // 16, fill_z, 0)

    @pl.when(s < 10)
    def _():
        pltpu.sync_copy(zbuf.at[pl.ds(0, 1000)],
                        deg_sh.at[pl.ds(s * 1000, 1000)])
    plsc.subcore_barrier()

    nrows = dst2_hbm.shape[0]
    half = nrows // NC
    lo = c * half

    hi = lo + half

    def step(ci, carry):
        r0 = lo + (2 * ci) * NS + s
        r1 = r0 + NS

        @pl.when(r0 < hi)
        def _():
            pltpu.sync_copy(dst2_hbm.at[r0], idxbuf)
            g0 = pltpu.async_copy(onesbuf, deg_sh.at[idxbuf], dsem0, add=True)

            @pl.when(r1 < hi)
            def _():
                pltpu.sync_copy(dst2_hbm.at[r1], idxbuf1)
                pltpu.async_copy(onesbuf, deg_sh.at[idxbuf1], dsem1,
                                 add=True).wait()
            g0.wait()
        return carry
    lax.fori_loop(0, (half + 2 * NS - 1) // (2 * NS), step, 0)
    plsc.subcore_barrier()

    @pl.when(s < 10)
    def _():
        pltpu.sync_copy(deg_sh.at[pl.ds(s * 1000, 1000)],
                        zbuf.at[pl.ds(0, 1000)])

        @pl.when(c == 0)
        def _():
            pltpu.sync_copy(zbuf.at[pl.ds(0, 1000)],
                            deg0_out.at[pl.ds(s * 1000, 1000)])

        @pl.when(c == 1)
        def _():
            pltpu.sync_copy(zbuf.at[pl.ds(0, 1000)],
                            deg1_out.at[pl.ds(s * 1000, 1000)])


def _sc_edge_body(y0_hbm, y1_hbm, sd_hbm, h0_out, h1_out,
                  sdbs, rowss, h_sh, gsems, ssems):
    c = lax.axis_index("c")
    s = lax.axis_index("s")
    nbuf = len(sdbs)
    rows0 = rowss[0]

    def zr(i, carry):
        rows0[i // 8, pl.ds((i % 8) * 16, 16)] = jnp.zeros((16,), jnp.float32)
        return carry
    lax.fori_loop(0, rows0.shape[0] * 8, zr, 0)

    @pl.when(s < 10)
    def _():
        sems = gsems + ssems
        zs = []
        for k in range(8):
            nr = 128 if k < 7 else 104
            zs.append(pltpu.async_copy(
                rows0.at[pl.ds(0, nr), :],
                h_sh.at[pl.ds(s * 1000 + k * 128, nr), :], sems[k % 6]))
        for z in zs:
            z.wait()
    plsc.subcore_barrier()

    nrows = sd_hbm.shape[0]

    def run(y_hbm):
        per = (nrows - s + NS - 1) // NS
        groups = per // nbuf

        def step(ci, carry):
            r0 = (nbuf * ci) * NS + s
            gs = []
            for b in range(nbuf):
                pltpu.sync_copy(sd_hbm.at[r0 + b * NS], sdbs[b])
                gs.append(pltpu.async_copy(y_hbm.at[sdbs[b].at[0]], rowss[b],
                                           gsems[b]))
            ss = []
            for b in range(nbuf):
                gs[b].wait()
                ss.append(pltpu.async_copy(rowss[b], h_sh.at[sdbs[b].at[1]],
                                           ssems[b], add=True))
            for b in range(nbuf):
                ss[b].wait()
            return carry
        lax.fori_loop(0, groups, step, 0)

        def tail(ti, carry):
            r = ti * NS + s
            pltpu.sync_copy(sd_hbm.at[r], sdbs[0])
            pltpu.async_copy(y_hbm.at[sdbs[0].at[0]], rowss[0],
                             gsems[0]).wait()
            pltpu.sync_copy(rowss[0], h_sh.at[sdbs[0].at[1]], add=True)
            return carry
        lax.fori_loop(groups * nbuf, per, tail, 0)

    @pl.when(c == 0)
    def _():
        run(y0_hbm)

    @pl.when(c == 1)
    def _():
        run(y1_hbm)
    plsc.subcore_barrier()

    @pl.when(s < 10)
    def _():
        for k in range(8):
            nr = 128 if k < 7 else 104
            base = pl.ds(s * 1000 + k * 128, nr)
            pltpu.sync_copy(h_sh.at[base, :], rows0.at[pl.ds(0, nr), :])

            @pl.when(c == 0)
            def _():
                pltpu.sync_copy(rows0.at[pl.ds(0, nr), :], h0_out.at[base, :])

            @pl.when(c == 1)
            def _():
                pltpu.sync_copy(rows0.at[pl.ds(0, nr), :], h1_out.at[base, :])


def kernel(x, edge_index, pool_p, gru_W_ih, gru_W_hh, gru_b_ih, gru_b_hh,
           W0, lin_W, lin_b):
    n, d = x.shape
    e = edge_index.shape[1]
    h = d // 2
    bn = 1000
    grid = n // bn

    dst2 = edge_index[1].reshape(e // 128, 128)
    sd2 = edge_index.reshape(2, e // 128, 128).transpose(1, 0, 2)
    mesh = plsc.VectorSubcoreMesh(core_axis_name="c", subcore_axis_name="s")

    deg0, deg1 = pl.kernel(
        _sc_deg_body,
        out_type=[jax.ShapeDtypeStruct((n,), jnp.float32),
                  jax.ShapeDtypeStruct((n,), jnp.float32)],
        mesh=mesh,
        scratch_types=[pltpu.VMEM((128,), jnp.int32),
                       pltpu.VMEM((128,), jnp.int32),
                       pltpu.VMEM((128,), jnp.float32),
                       pltpu.VMEM((1008,), jnp.float32),
                       pltpu.VMEM_SHARED((n,), jnp.float32),
                       pltpu.SemaphoreType.DMA,
                       pltpu.SemaphoreType.DMA],
    )(dst2)

    scores = pl.pallas_call(
        _score_body,
        grid=(grid,),
        in_specs=[pl.BlockSpec((bn, d), lambda i: (i, 0)),
                  pl.BlockSpec((1, d), lambda i: (0, 0))],
        out_specs=pl.BlockSpec((bn, 1), lambda i: (i, 0)),
        out_shape=jax.ShapeDtypeStruct((n, 1), jnp.float32),
    )(x, pool_p.reshape(1, d))

    vals, perm = lax.top_k(scores[:, 0], d)

    xg = pl.kernel(
        _sc_gather_rows_body,
        out_type=jax.ShapeDtypeStruct((d, d), jnp.float32),
        mesh=mesh,
        scratch_types=[pltpu.VMEM((d // (NC * NS),), jnp.int32),
                       pltpu.VMEM((d // (NC * NS), d), jnp.float32),
                       pltpu.SemaphoreType.DMA],
    )(x, perm.astype(jnp.int32))

    y0, y1, dinv = pl.pallas_call(
        _xw_body,
        grid=(grid,),
        in_specs=[pl.BlockSpec((bn, d), lambda i: (i, 0)),
                  pl.BlockSpec((d, d), lambda i: (0, 0)),
                  pl.BlockSpec((d, 1), lambda i: (0, 0)),
                  pl.BlockSpec((d, d), lambda i: (0, 0)),
                  pl.BlockSpec((3 * d, d), lambda i: (0, 0)),
                  pl.BlockSpec((3 * d, d), lambda i: (0, 0)),
                  pl.BlockSpec((1, 3 * d), lambda i: (0, 0)),
                  pl.BlockSpec((1, 3 * d), lambda i: (0, 0)),
                  pl.BlockSpec((bn, 1), lambda i: (i, 0)),
                  pl.BlockSpec((bn, 1), lambda i: (i, 0))],
        out_specs=[pl.BlockSpec((bn, h), lambda i: (i, 0)),
                   pl.BlockSpec((bn, h), lambda i: (i, 0)),
                   pl.BlockSpec((bn, 1), lambda i: (i, 0))],
        out_shape=[jax.ShapeDtypeStruct((n, h), jnp.float32),
                   jax.ShapeDtypeStruct((n, h), jnp.float32),
                   jax.ShapeDtypeStruct((n, 1), jnp.float32)],
        scratch_shapes=[pltpu.VMEM((d, d), jnp.float32)],
    )(x, xg, vals.reshape(d, 1), W0, gru_W_ih, gru_W_hh,
      gru_b_ih.reshape(1, 3 * d), gru_b_hh.reshape(1, 3 * d),
      deg0.reshape(n, 1), deg1.reshape(n, 1))

    h0, h1 = pl.kernel(
        _sc_edge_body,
        out_type=[jax.ShapeDtypeStruct((n, h), jnp.float32),
                  jax.ShapeDtypeStruct((n, h), jnp.float32)],
        mesh=mesh,
        scratch_types=[[pltpu.VMEM((2, 128), jnp.int32) for _ in range(3)],
                       [pltpu.VMEM((128, h), jnp.float32) for _ in range(3)],
                       pltpu.VMEM_SHARED((n, h), jnp.float32),
                       [pltpu.SemaphoreType.DMA for _ in range(3)],
                       [pltpu.SemaphoreType.DMA for _ in range(3)]],
    )(y0, y1, sd2)

    out = pl.pallas_call(
        _out_body,
        grid=(grid,),
        in_specs=[pl.BlockSpec((bn, h), lambda i: (i, 0)),
                  pl.BlockSpec((bn, h), lambda i: (i, 0)),
                  pl.BlockSpec((bn, h), lambda i: (i, 0)),
                  pl.BlockSpec((bn, h), lambda i: (i, 0)),
                  pl.BlockSpec((bn, 1), lambda i: (i, 0)),
                  pl.BlockSpec((d, d), lambda i: (0, 0)),
                  pl.BlockSpec((1, d), lambda i: (0, 0))],
        out_specs=pl.BlockSpec((bn, d), lambda i: (i, 0)),
        out_shape=jax.ShapeDtypeStruct((n, d), jnp.float32),
    )(h0, h1, y0, y1, dinv, lin_W, lin_b.reshape(1, d))
    return out


# speedup vs baseline: 1.0411x; 1.0411x over previous
import jax
import jax.numpy as jnp
from jax import lax
from jax.experimental import pallas as pl
from jax.experimental.pallas import tpu as pltpu
from jax.experimental.pallas import tpu_sc as plsc

NC = 2
NS = 16


def _score_body(x_ref, p_ref, o_ref):
    p = p_ref[0, :]
    xb = x_ref[...].astype(jnp.bfloat16)
    pb = p.astype(jnp.bfloat16)[:, None]
    s = lax.dot_general(xb, pb, (((1,), (0,)), ((), ())),
                        preferred_element_type=jnp.float32)
    o_ref[...] = s * lax.rsqrt(jnp.sum(p * p))


def _gru_w(xg, vals, w0, wih, whh, bih, bhh):
    d = w0.shape[1]
    xt = xg * jnp.tanh(vals)
    gi = lax.dot_general(xt, wih, (((1,), (1,)), ((), ())),
                         preferred_element_type=jnp.float32, precision=lax.Precision.HIGHEST) + bih
    gh = lax.dot_general(w0, whh, (((1,), (1,)), ((), ())),
                         preferred_element_type=jnp.float32, precision=lax.Precision.HIGHEST) + bhh
    r = jax.nn.sigmoid(gi[:, :d] + gh[:, :d])
    z = jax.nn.sigmoid(gi[:, d:2 * d] + gh[:, d:2 * d])
    n = jnp.tanh(gi[:, 2 * d:] + r * gh[:, 2 * d:])
    return (1.0 - z) * n + z * w0


def _xw_body(x_ref, xg_ref, vals_ref, w0_ref, wih_ref, whh_ref, bih_ref,
             bhh_ref, d0_ref, d1_ref, y0_ref, y1_ref, dinv_ref, w_scr):
    h = y0_ref.shape[1]

    @pl.when(pl.program_id(0) == 0)
    def _():
        w_scr[...] = _gru_w(xg_ref[...], vals_ref[...], w0_ref[...],
                            wih_ref[...], whh_ref[...], bih_ref[...],
                            bhh_ref[...])

    deg = d0_ref[...] + d1_ref[...] + 1.0
    dinv = lax.rsqrt(deg)
    xw = jnp.dot(x_ref[...], w_scr[...], preferred_element_type=jnp.float32, precision=lax.Precision.HIGHEST)
    y = xw * dinv
    y0_ref[...] = y[:, :h]
    y1_ref[...] = y[:, h:]
    dinv_ref[...] = dinv


def _out_body(h0_ref, h1_ref, y0_ref, y1_ref, dinv_ref, lw_ref, lb_ref,
              o_ref):
    dinv = dinv_ref[...]
    pre = jnp.concatenate(
        [h0_ref[...] + y0_ref[...], h1_ref[...] + y1_ref[...]], axis=1)
    r = jnp.maximum(pre * dinv, 0.0)
    o_ref[...] = lax.dot_general(r, lw_ref[...], (((1,), (1,)), ((), ())),
                                 preferred_element_type=jnp.float32, precision=lax.Precision.HIGHEST) + lb_ref[...]


def _sc_gather_rows_body(x_hbm, perm_hbm, xg_out, permbuf, rowsbuf, gsem):
    c = lax.axis_index("c")
    s = lax.axis_index("s")
    wid = c * NS + s
    k_per = perm_hbm.shape[0] // (NC * NS)
    pltpu.sync_copy(perm_hbm.at[pl.ds(wid * k_per, k_per)], permbuf)
    pltpu.async_copy(x_hbm.at[permbuf], rowsbuf, gsem).wait()
    pltpu.sync_copy(rowsbuf, xg_out.at[pl.ds(wid * k_per, k_per), :])


def _sc_deg_body(dst2_hbm, deg0_out, deg1_out,
                 idxbuf, idxbuf1, onesbuf, zbuf, deg_sh, dsem0, dsem1):
    c = lax.axis_index("c")
    s = lax.axis_index("s")

    def fill_ones(i, carry):
        onesbuf[pl.ds(i * 16, 16)] = jnp.full((16,), 1.0, jnp.float32)
        return carry
    lax.fori_loop(0, onesbuf.shape[0] // 16, fill_ones, 0)

    def fill_z(i, carry):
        zbuf[pl.ds(i * 16, 16)] = jnp.zeros((16,), jnp.float32)
        return carry
    lax.fori_loop(0, zbuf.shape[0] // 16, fill_z, 0)

    @pl.when(s < 10)
    def _():
        pltpu.sync_copy(zbuf.at[pl.ds(0, 1000)],
                        deg_sh.at[pl.ds(s * 1000, 1000)])
    plsc.subcore_barrier()

    nrows = dst2_hbm.shape[0]
    half = nrows // NC
    lo = c * half

    hi = lo + half

    def step(ci, carry):
        r0 = lo + (2 * ci) * NS + s
        r1 = r0 + NS

        @pl.when(r0 < hi)
        def _():
            pltpu.sync_copy(dst2_hbm.at[r0], idxbuf)
            g0 = pltpu.async_copy(onesbuf, deg_sh.at[idxbuf], dsem0, add=True)

            @pl.when(r1 < hi)
            def _():
                pltpu.sync_copy(dst2_hbm.at[r1], idxbuf1)
                pltpu.async_copy(onesbuf, deg_sh.at[idxbuf1], dsem1,
                                 add=True).wait()
            g0.wait()
        return carry
    lax.fori_loop(0, (half + 2 * NS - 1) // (2 * NS), step, 0)
    plsc.subcore_barrier()

    @pl.when(s < 10)
    def _():
        pltpu.sync_copy(deg_sh.at[pl.ds(s * 1000, 1000)],
                        zbuf.at[pl.ds(0, 1000)])

        @pl.when(c == 0)
        def _():
            pltpu.sync_copy(zbuf.at[pl.ds(0, 1000)],
                            deg0_out.at[pl.ds(s * 1000, 1000)])

        @pl.when(c == 1)
        def _():
            pltpu.sync_copy(zbuf.at[pl.ds(0, 1000)],
                            deg1_out.at[pl.ds(s * 1000, 1000)])


def _sc_edge_body(y0_hbm, y1_hbm, sd_hbm, h0_out, h1_out,
                  sdbs, rowss, h_sh, gsems, ssems):
    c = lax.axis_index("c")
    s = lax.axis_index("s")
    nbuf = len(sdbs)
    rows0 = rowss[0]

    def zr(i, carry):
        rows0[i // 8, pl.ds((i % 8) * 16, 16)] = jnp.zeros((16,), jnp.float32)
        return carry
    lax.fori_loop(0, rows0.shape[0] * 8, zr, 0)

    @pl.when(s < 10)
    def _():
        sems = gsems + ssems
        zs = []
        for k in range(8):
            nr = 128 if k < 7 else 104
            zs.append(pltpu.async_copy(
                rows0.at[pl.ds(0, nr), :],
                h_sh.at[pl.ds(s * 1000 + k * 128, nr), :], sems[k % 6]))
        for z in zs:
            z.wait()
    plsc.subcore_barrier()

    nrows = sd_hbm.shape[0]

    def run(y_hbm):
        per = (nrows - s + NS - 1) // NS
        groups = per // nbuf

        last = nbuf - 1

        def step(ci, carry):
            r0 = (nbuf * ci) * NS + s
            gs = []
            for b in range(nbuf):
                if b == last:
                    @pl.when(ci > 0)
                    def _():
                        pltpu.make_async_copy(rowss[last],
                                              h_sh.at[sdbs[last].at[1]],
                                              ssems[last]).wait()
                pltpu.sync_copy(sd_hbm.at[r0 + b * NS], sdbs[b])
                gs.append(pltpu.async_copy(y_hbm.at[sdbs[b].at[0]], rowss[b],
                                           gsems[b]))
            ss = []
            for b in range(nbuf):
                gs[b].wait()
                ss.append(pltpu.async_copy(rowss[b], h_sh.at[sdbs[b].at[1]],
                                           ssems[b], add=True))
            for b in range(nbuf - 1):
                ss[b].wait()
            return carry
        lax.fori_loop(0, groups, step, 0)

        @pl.when(groups > 0)
        def _():
            pltpu.make_async_copy(rowss[last], h_sh.at[sdbs[last].at[1]],
                                  ssems[last]).wait()

        def tail(ti, carry):
            r = ti * NS + s
            pltpu.sync_copy(sd_hbm.at[r], sdbs[0])
            pltpu.async_copy(y_hbm.at[sdbs[0].at[0]], rowss[0],
                             gsems[0]).wait()
            pltpu.sync_copy(rowss[0], h_sh.at[sdbs[0].at[1]], add=True)
            return carry
        lax.fori_loop(groups * nbuf, per, tail, 0)

    @pl.when(c == 0)
    def _():
        run(y0_hbm)

    @pl.when(c == 1)
    def _():
        run(y1_hbm)
    plsc.subcore_barrier()

    @pl.when(s < 10)
    def _():
        for k in range(8):
            nr = 128 if k < 7 else 104
            base = pl.ds(s * 1000 + k * 128, nr)
            pltpu.sync_copy(h_sh.at[base, :], rows0.at[pl.ds(0, nr), :])

            @pl.when(c == 0)
            def _():
                pltpu.sync_copy(rows0.at[pl.ds(0, nr), :], h0_out.at[base, :])

            @pl.when(c == 1)
            def _():
                pltpu.sync_copy(rows0.at[pl.ds(0, nr), :], h1_out.at[base, :])


def kernel(x, edge_index, pool_p, gru_W_ih, gru_W_hh, gru_b_ih, gru_b_hh,
           W0, lin_W, lin_b):
    n, d = x.shape
    e = edge_index.shape[1]
    h = d // 2
    bn = 1000
    grid = n // bn

    dst2 = edge_index[1].reshape(e // 128, 128)
    sd2 = edge_index.reshape(2, e // 128, 128).transpose(1, 0, 2)
    mesh = plsc.VectorSubcoreMesh(core_axis_name="c", subcore_axis_name="s")

    deg0, deg1 = pl.kernel(
        _sc_deg_body,
        out_type=[jax.ShapeDtypeStruct((n,), jnp.float32),
                  jax.ShapeDtypeStruct((n,), jnp.float32)],
        mesh=mesh,
        scratch_types=[pltpu.VMEM((128,), jnp.int32),
                       pltpu.VMEM((128,), jnp.int32),
                       pltpu.VMEM((128,), jnp.float32),
                       pltpu.VMEM((1008,), jnp.float32),
                       pltpu.VMEM_SHARED((n,), jnp.float32),
                       pltpu.SemaphoreType.DMA,
                       pltpu.SemaphoreType.DMA],
    )(dst2)

    scores = pl.pallas_call(
        _score_body,
        grid=(grid,),
        in_specs=[pl.BlockSpec((bn, d), lambda i: (i, 0)),
                  pl.BlockSpec((1, d), lambda i: (0, 0))],
        out_specs=pl.BlockSpec((bn, 1), lambda i: (i, 0)),
        out_shape=jax.ShapeDtypeStruct((n, 1), jnp.float32),
    )(x, pool_p.reshape(1, d))

    vals, perm = lax.top_k(scores[:, 0], d)

    xg = pl.kernel(
        _sc_gather_rows_body,
        out_type=jax.ShapeDtypeStruct((d, d), jnp.float32),
        mesh=mesh,
        scratch_types=[pltpu.VMEM((d // (NC * NS),), jnp.int32),
                       pltpu.VMEM((d // (NC * NS), d), jnp.float32),
                       pltpu.SemaphoreType.DMA],
    )(x, perm.astype(jnp.int32))

    y0, y1, dinv = pl.pallas_call(
        _xw_body,
        grid=(grid,),
        in_specs=[pl.BlockSpec((bn, d), lambda i: (i, 0)),
                  pl.BlockSpec((d, d), lambda i: (0, 0)),
                  pl.BlockSpec((d, 1), lambda i: (0, 0)),
                  pl.BlockSpec((d, d), lambda i: (0, 0)),
                  pl.BlockSpec((3 * d, d), lambda i: (0, 0)),
                  pl.BlockSpec((3 * d, d), lambda i: (0, 0)),
                  pl.BlockSpec((1, 3 * d), lambda i: (0, 0)),
                  pl.BlockSpec((1, 3 * d), lambda i: (0, 0)),
                  pl.BlockSpec((bn, 1), lambda i: (i, 0)),
                  pl.BlockSpec((bn, 1), lambda i: (i, 0))],
        out_specs=[pl.BlockSpec((bn, h), lambda i: (i, 0)),
                   pl.BlockSpec((bn, h), lambda i: (i, 0)),
                   pl.BlockSpec((bn, 1), lambda i: (i, 0))],
        out_shape=[jax.ShapeDtypeStruct((n, h), jnp.float32),
                   jax.ShapeDtypeStruct((n, h), jnp.float32),
                   jax.ShapeDtypeStruct((n, 1), jnp.float32)],
        scratch_shapes=[pltpu.VMEM((d, d), jnp.float32)],
    )(x, xg, vals.reshape(d, 1), W0, gru_W_ih, gru_W_hh,
      gru_b_ih.reshape(1, 3 * d), gru_b_hh.reshape(1, 3 * d),
      deg0.reshape(n, 1), deg1.reshape(n, 1))

    h0, h1 = pl.kernel(
        _sc_edge_body,
        out_type=[jax.ShapeDtypeStruct((n, h), jnp.float32),
                  jax.ShapeDtypeStruct((n, h), jnp.float32)],
        mesh=mesh,
        scratch_types=[[pltpu.VMEM((2, 128), jnp.int32) for _ in range(3)],
                       [pltpu.VMEM((128, h), jnp.float32) for _ in range(3)],
                       pltpu.VMEM_SHARED((n, h), jnp.float32),
                       [pltpu.SemaphoreType.DMA for _ in range(3)],
                       [pltpu.SemaphoreType.DMA for _ in range(3)]],
    )(y0, y1, sd2)

    out = pl.pallas_call(
        _out_body,
        grid=(grid,),
        in_specs=[pl.BlockSpec((bn, h), lambda i: (i, 0)),
                  pl.BlockSpec((bn, h), lambda i: (i, 0)),
                  pl.BlockSpec((bn, h), lambda i: (i, 0)),
                  pl.BlockSpec((bn, h), lambda i: (i, 0)),
                  pl.BlockSpec((bn, 1), lambda i: (i, 0)),
                  pl.BlockSpec((d, d), lambda i: (0, 0)),
                  pl.BlockSpec((1, d), lambda i: (0, 0))],
        out_specs=pl.BlockSpec((bn, d), lambda i: (i, 0)),
        out_shape=jax.ShapeDtypeStruct((n, d), jnp.float32),
    )(h0, h1, y0, y1, dinv, lin_W, lin_b.reshape(1, d))
    return out
